# baseline (device time: 979517 ns/iter reference)
import jax
import jax.numpy as jnp
from jax import lax
from jax.experimental import pallas as pl
from jax.experimental.pallas import tpu as pltpu

GX, GZ = 2, 4
GROUP = 8
K = 2048
MOUT = 2048
PM = MOUT // GX
F = 8192
CB = F // GROUP
S = 8
SC = CB // S
H = GROUP // 2


def _ring_coords(q):
    y = jnp.where(q < GZ, 0, 1)
    z = jnp.where(q < GZ, q, 2 * GZ - 1 - q)
    return y, z


def kernel(x, dy):
    def body(x_ref, dy_hbm, out_ref, stage, dyb, po, pm, xrv,
             copy_sem, store_sems, xsend_sems, xrecv_sems,
             cw_send, cw_recv, ccw_send, ccw_recv, fin_cw, fin_ccw):
        my_x = lax.axis_index("x")
        my_y = lax.axis_index("y")
        my_z = lax.axis_index("z")
        p = jnp.where(my_y == 0, my_z, 2 * GZ - 1 - my_z)
        right_y, right_z = _ring_coords((p + 1) % GROUP)
        left_y, left_z = _ring_coords((p - 1) % GROUP)
        other_x = 1 - my_x
        right_dev = (my_x, right_y, right_z)
        left_dev = (my_x, left_y, left_z)

        barrier = pltpu.get_barrier_semaphore()
        for nbr in ((other_x, my_y, my_z), right_dev, left_dev):
            pl.semaphore_signal(barrier, inc=1, device_id=nbr,
                                device_id_type=pl.DeviceIdType.MESH)
        pl.semaphore_wait(barrier, 3)

        cp = pltpu.make_async_copy(
            dy_hbm.at[:, pl.ds(p * CB, CB)], dyb, copy_sem)
        cp.start()

        c0 = (((0,), (0,)), ((), ()))
        x_other = x_ref[:, pl.ds(other_x * PM, PM)]
        x_mine = x_ref[:, pl.ds(my_x * PM, PM)]

        def ring_rdma(direction, h, s, orig):
            send, recv, dev = (
                (cw_send, cw_recv, right_dev) if direction == 0
                else (ccw_send, ccw_recv, left_dev))
            return pltpu.make_async_remote_copy(
                src_ref=stage.at[orig, s],
                dst_ref=stage.at[orig, s],
                send_sem=send.at[h * S + s],
                recv_sem=recv.at[h * S + s],
                device_id=dev,
                device_id_type=pl.DeviceIdType.MESH,
            )

        ring = {}
        xrs = []
        sts = []

        def finish_sub(s):
            sls = pl.ds(s * SC, SC)
            xrs[s].wait()
            pm[:, sls] = pm[:, sls] + xrv[:, sls]
            st = pltpu.make_async_copy(
                pm.at[:, sls], out_ref.at[:, pl.ds(p * CB + s * SC, SC)],
                store_sems.at[s])
            st.start()
            sts.append(st)
            for d in (0, 1):
                ring[(d, 0, s)] = pltpu.make_async_remote_copy(
                    src_ref=pm.at[:, sls],
                    dst_ref=stage.at[p, s],
                    send_sem=(cw_send if d == 0 else ccw_send).at[s],
                    recv_sem=(cw_recv if d == 0 else ccw_recv).at[s],
                    device_id=right_dev if d == 0 else left_dev,
                    device_id_type=pl.DeviceIdType.MESH,
                )
                ring[(d, 0, s)].start()

        cp.wait()
        for s in range(S):
            sls = pl.ds(s * SC, SC)
            po[:, sls] = lax.dot_general(
                x_other, dyb[:, sls],
                dimension_numbers=c0, preferred_element_type=jnp.float32)
            xr = pltpu.make_async_remote_copy(
                src_ref=po.at[:, sls],
                dst_ref=xrv.at[:, sls],
                send_sem=xsend_sems.at[s],
                recv_sem=xrecv_sems.at[s],
                device_id=(other_x, my_y, my_z),
                device_id_type=pl.DeviceIdType.MESH,
            )
            xr.start()
            xrs.append(xr)
            pm[:, sls] = lax.dot_general(
                x_mine, dyb[:, sls],
                dimension_numbers=c0, preferred_element_type=jnp.float32)
            if s >= 1:
                finish_sub(s - 1)
        finish_sub(S - 1)

        fins = []

        def drain(direction, h, s, orig):
            fsem = (fin_cw if direction == 0 else fin_ccw).at[h * S + s]
            fc = pltpu.make_async_copy(
                stage.at[orig, s],
                out_ref.at[:, pl.ds(orig * CB + s * SC, SC)],
                fsem)
            fc.start()
            fins.append(fc)

        for h in range(1, H):
            orig_cw = (p - h) % GROUP
            orig_ccw = (p + h) % GROUP
            cw_subs = range(S) if h < H - 1 else range(S // 2)
            ccw_subs = range(S) if h < H - 1 else range(S // 2, S)
            for s in range(S):
                ring[(0, h - 1, s)].wait()
                ring[(1, h - 1, s)].wait()
                if s in cw_subs:
                    ring[(0, h, s)] = ring_rdma(0, h, s, orig_cw)
                    ring[(0, h, s)].start()
                if s in ccw_subs:
                    ring[(1, h, s)] = ring_rdma(1, h, s, orig_ccw)
                    ring[(1, h, s)].start()
                drain(0, h - 1, s, orig_cw)
                drain(1, h - 1, s, orig_ccw)
        for s in range(S // 2):
            ring[(0, H - 1, s)].wait()
            drain(0, H - 1, s, (p - H) % GROUP)
        for s in range(S // 2, S):
            ring[(1, H - 1, s)].wait()
            drain(1, H - 1, s, (p + H) % GROUP)
        for st in sts:
            st.wait()
        for fc in fins:
            fc.wait()

    out_shape = (
        jax.ShapeDtypeStruct((PM, F), jnp.float32),
        jax.ShapeDtypeStruct((GROUP, S, PM, SC), jnp.float32),
    )
    out, _ = pl.pallas_call(
        body,
        out_shape=out_shape,
        in_specs=[
            pl.BlockSpec(memory_space=pltpu.VMEM),
            pl.BlockSpec(memory_space=pltpu.MemorySpace.HBM),
        ],
        out_specs=(
            pl.BlockSpec(memory_space=pltpu.MemorySpace.HBM),
            pl.BlockSpec(memory_space=pltpu.MemorySpace.HBM),
        ),
        scratch_shapes=[
            pltpu.VMEM((K, CB), jnp.float32),
            pltpu.VMEM((PM, CB), jnp.float32),
            pltpu.VMEM((PM, CB), jnp.float32),
            pltpu.VMEM((PM, CB), jnp.float32),
            pltpu.SemaphoreType.DMA,
            pltpu.SemaphoreType.DMA((S,)),
            pltpu.SemaphoreType.DMA((S,)),
            pltpu.SemaphoreType.DMA((S,)),
            pltpu.SemaphoreType.DMA((H * S,)),
            pltpu.SemaphoreType.DMA((H * S,)),
            pltpu.SemaphoreType.DMA((H * S,)),
            pltpu.SemaphoreType.DMA((H * S,)),
            pltpu.SemaphoreType.DMA((H * S,)),
            pltpu.SemaphoreType.DMA((H * S,)),
        ],
        compiler_params=pltpu.CompilerParams(
            collective_id=0,
            vmem_limit_bytes=58 * 1024 * 1024,
        ),
    )(x, dy)
    return out


# device time: 206305 ns/iter; 4.7479x vs baseline; 4.7479x over previous
import jax
import jax.numpy as jnp
from jax import lax
from jax.experimental import pallas as pl
from jax.experimental.pallas import tpu as pltpu

GX, GZ = 2, 4
GROUP = 8
K = 2048
MOUT = 2048
PM = MOUT // GX
F = 8192
CB = F // GROUP
S = 8
SC = CB // S
H = GROUP // 2


def _ring_coords(q):
    y = jnp.where(q < GZ, 0, 1)
    z = jnp.where(q < GZ, q, 2 * GZ - 1 - q)
    return y, z


def kernel(x, dy):
    def body(x_ref, dy_hbm, out_ref, dyb, po, pm, xrv,
             copy_sem, store_sems, xsend_sems, xrecv_sems,
             cw_send, cw_recv, ccw_send, ccw_recv):
        my_x = lax.axis_index("x")
        my_y = lax.axis_index("y")
        my_z = lax.axis_index("z")
        p = jnp.where(my_y == 0, my_z, 2 * GZ - 1 - my_z)
        right_y, right_z = _ring_coords((p + 1) % GROUP)
        left_y, left_z = _ring_coords((p - 1) % GROUP)
        other_x = 1 - my_x
        right_dev = (my_x, right_y, right_z)
        left_dev = (my_x, left_y, left_z)

        barrier = pltpu.get_barrier_semaphore()
        for nbr in ((other_x, my_y, my_z), right_dev, left_dev):
            pl.semaphore_signal(barrier, inc=1, device_id=nbr,
                                device_id_type=pl.DeviceIdType.MESH)
        pl.semaphore_wait(barrier, 3)

        cp = pltpu.make_async_copy(
            dy_hbm.at[:, pl.ds(p * CB, CB)], dyb, copy_sem)
        cp.start()

        c0 = (((0,), (0,)), ((), ()))
        x_other = x_ref[:, pl.ds(other_x * PM, PM)]
        x_mine = x_ref[:, pl.ds(my_x * PM, PM)]

        def ring_rdma(direction, h, s, orig):
            sl = pl.ds(orig * CB + s * SC, SC)
            send, recv, dev = (
                (cw_send, cw_recv, right_dev) if direction == 0
                else (ccw_send, ccw_recv, left_dev))
            return pltpu.make_async_remote_copy(
                src_ref=out_ref.at[:, sl],
                dst_ref=out_ref.at[:, sl],
                send_sem=send.at[h * S + s],
                recv_sem=recv.at[h * S + s],
                device_id=dev,
                device_id_type=pl.DeviceIdType.MESH,
            )

        ring = {}
        xrs = []
        sts = []

        def finish_sub(s):
            xrs[s].wait()
            pm[s] = pm[s] + xrv[s]
            st = pltpu.make_async_copy(
                pm.at[s], out_ref.at[:, pl.ds(p * CB + s * SC, SC)],
                store_sems.at[s])
            st.start()
            sts.append(st)
            for d in (0, 1):
                ring[(d, 0, s)] = pltpu.make_async_remote_copy(
                    src_ref=pm.at[s],
                    dst_ref=out_ref.at[:, pl.ds(p * CB + s * SC, SC)],
                    send_sem=(cw_send if d == 0 else ccw_send).at[s],
                    recv_sem=(cw_recv if d == 0 else ccw_recv).at[s],
                    device_id=right_dev if d == 0 else left_dev,
                    device_id_type=pl.DeviceIdType.MESH,
                )
                ring[(d, 0, s)].start()

        cp.wait()
        for s in range(S):
            sls = pl.ds(s * SC, SC)
            po[s] = lax.dot_general(
                x_other, dyb[:, sls],
                dimension_numbers=c0, preferred_element_type=jnp.float32)
            xr = pltpu.make_async_remote_copy(
                src_ref=po.at[s],
                dst_ref=xrv.at[s],
                send_sem=xsend_sems.at[s],
                recv_sem=xrecv_sems.at[s],
                device_id=(other_x, my_y, my_z),
                device_id_type=pl.DeviceIdType.MESH,
            )
            xr.start()
            xrs.append(xr)
            pm[s] = lax.dot_general(
                x_mine, dyb[:, sls],
                dimension_numbers=c0, preferred_element_type=jnp.float32)
            if s >= 1:
                finish_sub(s - 1)
        finish_sub(S - 1)

        for h in range(1, H):
            orig_cw = (p - h) % GROUP
            orig_ccw = (p + h) % GROUP
            cw_subs = range(S) if h < H - 1 else range(S // 2)
            ccw_subs = range(S) if h < H - 1 else range(S // 2, S)
            for s in range(S):
                ring[(0, h - 1, s)].wait()
                ring[(1, h - 1, s)].wait()
                if s in cw_subs:
                    ring[(0, h, s)] = ring_rdma(0, h, s, orig_cw)
                    ring[(0, h, s)].start()
                if s in ccw_subs:
                    ring[(1, h, s)] = ring_rdma(1, h, s, orig_ccw)
                    ring[(1, h, s)].start()
        for s in range(S // 2):
            ring[(0, H - 1, s)].wait()
        for s in range(S // 2, S):
            ring[(1, H - 1, s)].wait()
        for st in sts:
            st.wait()

    out_shape = jax.ShapeDtypeStruct((PM, F), jnp.float32)
    return pl.pallas_call(
        body,
        out_shape=out_shape,
        in_specs=[
            pl.BlockSpec(memory_space=pltpu.VMEM),
            pl.BlockSpec(memory_space=pltpu.MemorySpace.HBM),
        ],
        out_specs=pl.BlockSpec(memory_space=pltpu.MemorySpace.HBM),
        scratch_shapes=[
            pltpu.VMEM((K, CB), jnp.float32),
            pltpu.VMEM((S, PM, SC), jnp.float32),
            pltpu.VMEM((S, PM, SC), jnp.float32),
            pltpu.VMEM((S, PM, SC), jnp.float32),
            pltpu.SemaphoreType.DMA,
            pltpu.SemaphoreType.DMA((S,)),
            pltpu.SemaphoreType.DMA((S,)),
            pltpu.SemaphoreType.DMA((S,)),
            pltpu.SemaphoreType.DMA((H * S,)),
            pltpu.SemaphoreType.DMA((H * S,)),
            pltpu.SemaphoreType.DMA((H * S,)),
            pltpu.SemaphoreType.DMA((H * S,)),
        ],
        compiler_params=pltpu.CompilerParams(
            collective_id=0,
            vmem_limit_bytes=58 * 1024 * 1024,
        ),
    )(x, dy)


# device time: 204582 ns/iter; 4.7879x vs baseline; 1.0084x over previous
import jax
import jax.numpy as jnp
from jax import lax
from jax.experimental import pallas as pl
from jax.experimental.pallas import tpu as pltpu

GX, GZ = 2, 4
GROUP = 8
K = 2048
MOUT = 2048
PM = MOUT // GX
F = 8192
CB = F // GROUP
S = 8
SC = CB // S
H = GROUP // 2


def _ring_coords(q):
    y = jnp.where(q < GZ, 0, 1)
    z = jnp.where(q < GZ, q, 2 * GZ - 1 - q)
    return y, z


def kernel(x, dy):
    def body(x_hbm, dy_hbm, out_ref, xob, xmb, dyb, po, pm, xrv,
             xo_sem, xm_sem, copy_sem, store_sems, xsend_sems, xrecv_sems,
             cw_send, cw_recv, ccw_send, ccw_recv):
        my_x = lax.axis_index("x")
        my_y = lax.axis_index("y")
        my_z = lax.axis_index("z")
        p = jnp.where(my_y == 0, my_z, 2 * GZ - 1 - my_z)
        right_y, right_z = _ring_coords((p + 1) % GROUP)
        left_y, left_z = _ring_coords((p - 1) % GROUP)
        other_x = 1 - my_x
        right_dev = (my_x, right_y, right_z)
        left_dev = (my_x, left_y, left_z)

        xoc = pltpu.make_async_copy(
            x_hbm.at[:, pl.ds(other_x * PM, PM)], xob, xo_sem)
        xoc.start()
        xmc = pltpu.make_async_copy(
            x_hbm.at[:, pl.ds(my_x * PM, PM)], xmb, xm_sem)
        xmc.start()

        barrier = pltpu.get_barrier_semaphore()
        for nbr in ((other_x, my_y, my_z), right_dev, left_dev):
            pl.semaphore_signal(barrier, inc=1, device_id=nbr,
                                device_id_type=pl.DeviceIdType.MESH)
        pl.semaphore_wait(barrier, 3)

        cp = pltpu.make_async_copy(
            dy_hbm.at[:, pl.ds(p * CB, CB)], dyb, copy_sem)
        cp.start()

        c0 = (((0,), (0,)), ((), ()))

        def ring_rdma(direction, h, s, orig):
            sl = pl.ds(orig * CB + s * SC, SC)
            send, recv, dev = (
                (cw_send, cw_recv, right_dev) if direction == 0
                else (ccw_send, ccw_recv, left_dev))
            return pltpu.make_async_remote_copy(
                src_ref=out_ref.at[:, sl],
                dst_ref=out_ref.at[:, sl],
                send_sem=send.at[h * S + s],
                recv_sem=recv.at[h * S + s],
                device_id=dev,
                device_id_type=pl.DeviceIdType.MESH,
            )

        ring = {}
        xrs = []
        sts = []

        def finish_sub(s):
            xrs[s].wait()
            pm[s] = pm[s] + xrv[s]
            st = pltpu.make_async_copy(
                pm.at[s], out_ref.at[:, pl.ds(p * CB + s * SC, SC)],
                store_sems.at[s])
            st.start()
            sts.append(st)
            for d in (0, 1):
                ring[(d, 0, s)] = pltpu.make_async_remote_copy(
                    src_ref=pm.at[s],
                    dst_ref=out_ref.at[:, pl.ds(p * CB + s * SC, SC)],
                    send_sem=(cw_send if d == 0 else ccw_send).at[s],
                    recv_sem=(cw_recv if d == 0 else ccw_recv).at[s],
                    device_id=right_dev if d == 0 else left_dev,
                    device_id_type=pl.DeviceIdType.MESH,
                )
                ring[(d, 0, s)].start()

        cp.wait()
        xoc.wait()
        xmc.wait()
        for s in range(S):
            sls = pl.ds(s * SC, SC)
            po[s] = lax.dot_general(
                xob[...], dyb[:, sls],
                dimension_numbers=c0, preferred_element_type=jnp.float32)
            xr = pltpu.make_async_remote_copy(
                src_ref=po.at[s],
                dst_ref=xrv.at[s],
                send_sem=xsend_sems.at[s],
                recv_sem=xrecv_sems.at[s],
                device_id=(other_x, my_y, my_z),
                device_id_type=pl.DeviceIdType.MESH,
            )
            xr.start()
            xrs.append(xr)
            pm[s] = lax.dot_general(
                xmb[...], dyb[:, sls],
                dimension_numbers=c0, preferred_element_type=jnp.float32)
            if s >= 1:
                finish_sub(s - 1)
        finish_sub(S - 1)

        for h in range(1, H):
            orig_cw = (p - h) % GROUP
            orig_ccw = (p + h) % GROUP
            cw_subs = range(S) if h < H - 1 else range(S // 2)
            ccw_subs = range(S) if h < H - 1 else range(S // 2, S)
            for s in range(S):
                ring[(0, h - 1, s)].wait()
                ring[(1, h - 1, s)].wait()
                if s in cw_subs:
                    ring[(0, h, s)] = ring_rdma(0, h, s, orig_cw)
                    ring[(0, h, s)].start()
                if s in ccw_subs:
                    ring[(1, h, s)] = ring_rdma(1, h, s, orig_ccw)
                    ring[(1, h, s)].start()
        for s in range(S // 2):
            ring[(0, H - 1, s)].wait()
        for s in range(S // 2, S):
            ring[(1, H - 1, s)].wait()
        for st in sts:
            st.wait()

    out_shape = jax.ShapeDtypeStruct((PM, F), jnp.float32)
    return pl.pallas_call(
        body,
        out_shape=out_shape,
        in_specs=[
            pl.BlockSpec(memory_space=pltpu.MemorySpace.HBM),
            pl.BlockSpec(memory_space=pltpu.MemorySpace.HBM),
        ],
        out_specs=pl.BlockSpec(memory_space=pltpu.MemorySpace.HBM),
        scratch_shapes=[
            pltpu.VMEM((K, PM), jnp.float32),
            pltpu.VMEM((K, PM), jnp.float32),
            pltpu.VMEM((K, CB), jnp.float32),
            pltpu.VMEM((S, PM, SC), jnp.float32),
            pltpu.VMEM((S, PM, SC), jnp.float32),
            pltpu.VMEM((S, PM, SC), jnp.float32),
            pltpu.SemaphoreType.DMA,
            pltpu.SemaphoreType.DMA,
            pltpu.SemaphoreType.DMA,
            pltpu.SemaphoreType.DMA((S,)),
            pltpu.SemaphoreType.DMA((S,)),
            pltpu.SemaphoreType.DMA((S,)),
            pltpu.SemaphoreType.DMA((H * S,)),
            pltpu.SemaphoreType.DMA((H * S,)),
            pltpu.SemaphoreType.DMA((H * S,)),
            pltpu.SemaphoreType.DMA((H * S,)),
        ],
        compiler_params=pltpu.CompilerParams(
            collective_id=0,
            vmem_limit_bytes=58 * 1024 * 1024,
        ),
    )(x, dy)


# device time: 204242 ns/iter; 4.7959x vs baseline; 1.0017x over previous
import jax
import jax.numpy as jnp
from jax import lax
from jax.experimental import pallas as pl
from jax.experimental.pallas import tpu as pltpu

GX, GZ = 2, 4
GROUP = 8
K = 2048
MOUT = 2048
PM = MOUT // GX
F = 8192
CB = F // GROUP
S = 8
SC = CB // S
H = GROUP // 2


def _ring_coords(q):
    y = jnp.where(q < GZ, 0, 1)
    z = jnp.where(q < GZ, q, 2 * GZ - 1 - q)
    return y, z


def kernel(x, dy):
    def body(x_hbm, dy_hbm, out_ref, xob, xmb, dyb, po, pm, xrv,
             xo_sem, xm_sem, copy_sem, store_sems, xsend_sems, xrecv_sems,
             cw_send, cw_recv, ccw_send, ccw_recv):
        my_x = lax.axis_index("x")
        my_y = lax.axis_index("y")
        my_z = lax.axis_index("z")
        p = jnp.where(my_y == 0, my_z, 2 * GZ - 1 - my_z)
        right_y, right_z = _ring_coords((p + 1) % GROUP)
        left_y, left_z = _ring_coords((p - 1) % GROUP)
        other_x = 1 - my_x
        right_dev = (my_x, right_y, right_z)
        left_dev = (my_x, left_y, left_z)

        xoc = pltpu.make_async_copy(
            x_hbm.at[:, pl.ds(other_x * PM, PM)], xob, xo_sem)
        xoc.start()
        xmc = pltpu.make_async_copy(
            x_hbm.at[:, pl.ds(my_x * PM, PM)], xmb, xm_sem)
        xmc.start()

        barrier = pltpu.get_barrier_semaphore()
        for nbr in ((other_x, my_y, my_z), right_dev, left_dev):
            pl.semaphore_signal(barrier, inc=1, device_id=nbr,
                                device_id_type=pl.DeviceIdType.MESH)
        pl.semaphore_wait(barrier, 3)

        cp = pltpu.make_async_copy(
            dy_hbm.at[:, pl.ds(p * CB, CB)], dyb, copy_sem)
        cp.start()

        c0 = (((0,), (0,)), ((), ()))

        def ring_rdma(direction, h, s, orig):
            sl = pl.ds(orig * CB + s * SC, SC)
            send, recv, dev = (
                (cw_send, cw_recv, right_dev) if direction == 0
                else (ccw_send, ccw_recv, left_dev))
            return pltpu.make_async_remote_copy(
                src_ref=out_ref.at[:, sl],
                dst_ref=out_ref.at[:, sl],
                send_sem=send.at[h * S + s],
                recv_sem=recv.at[h * S + s],
                device_id=dev,
                device_id_type=pl.DeviceIdType.MESH,
            )

        ring = {}
        xrs = []
        sts = []

        def finish_sub(s):
            xrs[s].wait()
            pm[s] = pm[s] + xrv[s]
            st = pltpu.make_async_copy(
                pm.at[s], out_ref.at[:, pl.ds(p * CB + s * SC, SC)],
                store_sems.at[s])
            st.start()
            sts.append(st)
            for d in (0, 1):
                ring[(d, 0, s)] = pltpu.make_async_remote_copy(
                    src_ref=pm.at[s],
                    dst_ref=out_ref.at[:, pl.ds(p * CB + s * SC, SC)],
                    send_sem=(cw_send if d == 0 else ccw_send).at[s],
                    recv_sem=(cw_recv if d == 0 else ccw_recv).at[s],
                    device_id=right_dev if d == 0 else left_dev,
                    device_id_type=pl.DeviceIdType.MESH,
                )
                ring[(d, 0, s)].start()

        cp.wait()
        xoc.wait()
        xmc.wait()
        for s in range(S):
            sls = pl.ds(s * SC, SC)
            po[s] = lax.dot_general(
                xob[...], dyb[:, sls],
                dimension_numbers=c0, preferred_element_type=jnp.float32)
            xr = pltpu.make_async_remote_copy(
                src_ref=po.at[s],
                dst_ref=xrv.at[s],
                send_sem=xsend_sems.at[s],
                recv_sem=xrecv_sems.at[s],
                device_id=(other_x, my_y, my_z),
                device_id_type=pl.DeviceIdType.MESH,
            )
            xr.start()
            xrs.append(xr)
            pm[s] = lax.dot_general(
                xmb[...], dyb[:, sls],
                dimension_numbers=c0, preferred_element_type=jnp.float32)
            if s >= 1:
                finish_sub(s - 1)
        finish_sub(S - 1)

        for h in range(1, H):
            orig_cw = (p - h) % GROUP
            orig_ccw = (p + h) % GROUP
            cw_subs = range(S) if h < H - 1 else range(S // 2)
            ccw_subs = range(S) if h < H - 1 else range(S // 2, S)
            for s in range(S):
                ring[(0, h - 1, s)].wait()
                ring[(1, h - 1, s)].wait()
                if s in cw_subs:
                    ring[(0, h, s)] = ring_rdma(0, h, s, orig_cw)
                    ring[(0, h, s)].start()
                if s in ccw_subs:
                    ring[(1, h, s)] = ring_rdma(1, h, s, orig_ccw)
                    ring[(1, h, s)].start()
        for s in range(S // 2):
            ring[(0, H - 1, s)].wait()
        for s in range(S // 2, S):
            ring[(1, H - 1, s)].wait()
        for st in sts:
            st.wait()

    out_shape = jax.ShapeDtypeStruct((PM, F), jnp.float32)
    return pl.pallas_call(
        body,
        out_shape=out_shape,
        in_specs=[
            pl.BlockSpec(memory_space=pl.MemorySpace.ANY),
            pl.BlockSpec(memory_space=pl.MemorySpace.ANY),
        ],
        out_specs=pl.BlockSpec(memory_space=pl.MemorySpace.ANY),
        scratch_shapes=[
            pltpu.VMEM((K, PM), jnp.float32),
            pltpu.VMEM((K, PM), jnp.float32),
            pltpu.VMEM((K, CB), jnp.float32),
            pltpu.VMEM((S, PM, SC), jnp.float32),
            pltpu.VMEM((S, PM, SC), jnp.float32),
            pltpu.VMEM((S, PM, SC), jnp.float32),
            pltpu.SemaphoreType.DMA,
            pltpu.SemaphoreType.DMA,
            pltpu.SemaphoreType.DMA,
            pltpu.SemaphoreType.DMA((S,)),
            pltpu.SemaphoreType.DMA((S,)),
            pltpu.SemaphoreType.DMA((S,)),
            pltpu.SemaphoreType.DMA((H * S,)),
            pltpu.SemaphoreType.DMA((H * S,)),
            pltpu.SemaphoreType.DMA((H * S,)),
            pltpu.SemaphoreType.DMA((H * S,)),
        ],
        compiler_params=pltpu.CompilerParams(
            collective_id=0,
            vmem_limit_bytes=58 * 1024 * 1024,
        ),
    )(x, dy)
